# Initial kernel scaffold; baseline (speedup 1.0000x reference)
import jax
import jax.numpy as jnp
from jax.experimental import pallas as pl


def kernel(x, W1, g1, b1, W2, g2, b2, W3, g3, b3, W4, g4, b4, W5, g5, b5, Wg, gg, bg, Wc1, gc1, bc1, Wc2, gc2, bc2, Wc3, bc3):
    raise NotImplementedError("write your pallas kernel here")


# speedup vs baseline: 2.6988x; 2.6988x over previous
import functools

import jax
import jax.numpy as jnp
from jax.experimental import pallas as pl

_K = 20
_EPS = 1e-5
_RB = 256
_NEG = -3.0e38

_INTERPRET = False


def _pcall(body, grid, in_specs, out_specs, out_shape):
    return pl.pallas_call(
        body,
        grid=grid,
        in_specs=in_specs,
        out_specs=out_specs,
        out_shape=out_shape,
        interpret=_INTERPRET,
    )


def _edge_pre_body(N, fb_ref, ff_ref, wnT_ref, wcmT_ref, y_ref, st_ref):
    b = pl.program_id(0)
    nb = pl.program_id(1)
    fb = fb_ref[0]
    ff = ff_ref[0]
    wnT = wnT_ref[...]
    wcmT = wcmT_ref[...]

    z = jnp.dot(ff, wnT, preferred_element_type=jnp.float32, precision=jax.lax.Precision.HIGHEST)
    cterm = jnp.dot(fb, wcmT, preferred_element_type=jnp.float32, precision=jax.lax.Precision.HIGHEST)

    sqf = jnp.sum(ff * ff, axis=1)[None, :]
    sqb = jnp.sum(fb * fb, axis=1, keepdims=True)
    inner = jax.lax.dot_general(fb, ff, (((1,), (1,)), ((), ())),
                                preferred_element_type=jnp.float32, precision=jax.lax.Precision.HIGHEST)
    pd = 2.0 * inner - sqb - sqf

    iota = jax.lax.broadcasted_iota(jnp.int32, (_RB, N), 1)
    rows = nb * _RB + jax.lax.broadcasted_iota(jnp.int32, (_RB, N), 0)
    pd = jnp.where(iota == rows, _NEG, pd)

    y0 = z_self = jnp.dot(fb, wnT, preferred_element_type=jnp.float32, precision=jax.lax.Precision.HIGHEST) + cterm
    y_ref[0, :, 0, :] = y0
    s = jnp.sum(y0, axis=0, keepdims=True)
    s2 = jnp.sum(y0 * y0, axis=0, keepdims=True)

    for t in range(1, _K):
        m = jnp.max(pd, axis=1, keepdims=True)
        eq = pd >= m
        idxv = jnp.min(jnp.where(eq, iota, N), axis=1, keepdims=True)
        onehot = iota == idxv
        pd = jnp.where(onehot, _NEG, pd)
        oh = jnp.where(onehot, 1.0, 0.0).astype(jnp.float32)
        nbr = jnp.dot(oh, z, preferred_element_type=jnp.float32, precision=jax.lax.Precision.HIGHEST)
        yt = nbr + cterm
        y_ref[0, :, t, :] = yt
        s = s + jnp.sum(yt, axis=0, keepdims=True)
        s2 = s2 + jnp.sum(yt * yt, axis=0, keepdims=True)

    @pl.when((b == 0) & (nb == 0))
    def _():
        st_ref[...] = jnp.zeros_like(st_ref)

    st_ref[0:1, :] += s
    st_ref[1:2, :] += s2


def _edge_pre(f, wnT, wcmT):
    B, N, C = f.shape
    O = wnT.shape[1]
    grid = (B, N // _RB)
    return _pcall(
        functools.partial(_edge_pre_body, N),
        grid,
        in_specs=[
            pl.BlockSpec((1, _RB, C), lambda b, nb: (b, nb, 0)),
            pl.BlockSpec((1, N, C), lambda b, nb: (b, 0, 0)),
            pl.BlockSpec((C, O), lambda b, nb: (0, 0)),
            pl.BlockSpec((C, O), lambda b, nb: (0, 0)),
        ],
        out_specs=[
            pl.BlockSpec((1, _RB, _K, O), lambda b, nb: (b, nb, 0, 0)),
            pl.BlockSpec((8, O), lambda b, nb: (0, 0)),
        ],
        out_shape=[
            jax.ShapeDtypeStruct((B, N, _K, O), jnp.float32),
            jax.ShapeDtypeStruct((8, O), jnp.float32),
        ],
    )(f, f, wnT, wcmT)


def _conv_stage_body(Kk, cnt, act, has_prev, has_bias,
                     *refs):
    i = 0
    y_ref = refs[i]; i += 1
    if has_prev:
        st_in = refs[i]; i += 1
        g_ref = refs[i]; i += 1
        bb_ref = refs[i]; i += 1
    wT_ref = refs[i]; i += 1
    if has_bias:
        bias_ref = refs[i]; i += 1
    yo_ref = refs[i]; i += 1
    so_ref = refs[i]; i += 1

    b = pl.program_id(0)
    nb = pl.program_id(1)

    if has_prev:
        mean = st_in[0:1, :] / cnt
        var = st_in[1:2, :] / cnt - mean * mean
        scale = g_ref[...] * jax.lax.rsqrt(var + _EPS)
        shift = bb_ref[...] - mean * scale

    wT = wT_ref[...]
    s = None
    for k in range(Kk):
        h = y_ref[0, :, k, :]
        if has_prev:
            h = h * scale + shift
            if act:
                h = jnp.where(h >= 0, h, 0.2 * h)
        yo = jnp.dot(h, wT, preferred_element_type=jnp.float32, precision=jax.lax.Precision.HIGHEST)
        if has_bias:
            yo = yo + bias_ref[...]
        yo_ref[0, :, k, :] = yo
        ds = jnp.sum(yo, axis=0, keepdims=True)
        ds2 = jnp.sum(yo * yo, axis=0, keepdims=True)
        s = ds if s is None else s + ds
        s2 = ds2 if k == 0 else s2 + ds2

    @pl.when((b == 0) & (nb == 0))
    def _():
        so_ref[...] = jnp.zeros_like(so_ref)

    so_ref[0:1, :] += s
    so_ref[1:2, :] += s2


def _conv_stage(y, stats_in, g, bb, wT, cnt, act=True, bias=None):
    B, N, Kk, Cin = y.shape
    Cout = wT.shape[1]
    grid = (B, N // _RB)
    has_prev = stats_in is not None
    has_bias = bias is not None
    in_specs = [pl.BlockSpec((1, _RB, Kk, Cin), lambda b, nb: (b, nb, 0, 0))]
    ops = [y]
    if has_prev:
        in_specs += [
            pl.BlockSpec((8, Cin), lambda b, nb: (0, 0)),
            pl.BlockSpec((1, Cin), lambda b, nb: (0, 0)),
            pl.BlockSpec((1, Cin), lambda b, nb: (0, 0)),
        ]
        ops += [stats_in, g.reshape(1, Cin), bb.reshape(1, Cin)]
    in_specs.append(pl.BlockSpec((Cin, Cout), lambda b, nb: (0, 0)))
    ops.append(wT)
    if has_bias:
        in_specs.append(pl.BlockSpec((1, Cout), lambda b, nb: (0, 0)))
        ops.append(bias.reshape(1, Cout))
    return _pcall(
        functools.partial(_conv_stage_body, Kk, cnt, act, has_prev, has_bias),
        grid,
        in_specs=in_specs,
        out_specs=[
            pl.BlockSpec((1, _RB, Kk, Cout), lambda b, nb: (b, nb, 0, 0)),
            pl.BlockSpec((8, Cout), lambda b, nb: (0, 0)),
        ],
        out_shape=[
            jax.ShapeDtypeStruct((B, N, Kk, Cout), jnp.float32),
            jax.ShapeDtypeStruct((8, Cout), jnp.float32),
        ],
    )(*ops)


def _bn_max_body(Kk, cnt, y_ref, st_ref, g_ref, b_ref, o_ref):
    mean = st_ref[0:1, :] / cnt
    var = st_ref[1:2, :] / cnt - mean * mean
    scale = g_ref[...] * jax.lax.rsqrt(var + _EPS)
    shift = b_ref[...] - mean * scale
    acc = None
    for k in range(Kk):
        h = y_ref[0, :, k, :] * scale + shift
        h = jnp.where(h >= 0, h, 0.2 * h)
        acc = h if acc is None else jnp.maximum(acc, h)
    o_ref[0] = acc


def _bn_max_k(y, stats, g, bb, cnt):
    B, N, Kk, C = y.shape
    grid = (B, N // _RB)
    return _pcall(
        functools.partial(_bn_max_body, Kk, cnt),
        grid,
        in_specs=[
            pl.BlockSpec((1, _RB, Kk, C), lambda b, nb: (b, nb, 0, 0)),
            pl.BlockSpec((8, C), lambda b, nb: (0, 0)),
            pl.BlockSpec((1, C), lambda b, nb: (0, 0)),
            pl.BlockSpec((1, C), lambda b, nb: (0, 0)),
        ],
        out_specs=pl.BlockSpec((1, _RB, C), lambda b, nb: (b, nb, 0)),
        out_shape=jax.ShapeDtypeStruct((B, N, C), jnp.float32),
    )(y, stats, g.reshape(1, C), bb.reshape(1, C))


def _bn_maxn_body(cnt, y_ref, st_ref, g_ref, b_ref, o_ref):
    nb = pl.program_id(1)
    mean = st_ref[0:1, :] / cnt
    var = st_ref[1:2, :] / cnt - mean * mean
    scale = g_ref[...] * jax.lax.rsqrt(var + _EPS)
    shift = b_ref[...] - mean * scale
    h = y_ref[0, :, 0, :] * scale + shift
    h = jnp.where(h >= 0, h, 0.2 * h)
    m = jnp.max(h, axis=0, keepdims=True)

    @pl.when(nb == 0)
    def _():
        o_ref[0] = m

    @pl.when(nb != 0)
    def _():
        o_ref[0] = jnp.maximum(o_ref[0], m)


def _bn_max_n(y, stats, g, bb, cnt):
    B, N, _, C = y.shape
    grid = (B, N // _RB)
    return _pcall(
        functools.partial(_bn_maxn_body, cnt),
        grid,
        in_specs=[
            pl.BlockSpec((1, _RB, 1, C), lambda b, nb: (b, nb, 0, 0)),
            pl.BlockSpec((8, C), lambda b, nb: (0, 0)),
            pl.BlockSpec((1, C), lambda b, nb: (0, 0)),
            pl.BlockSpec((1, C), lambda b, nb: (0, 0)),
        ],
        out_specs=pl.BlockSpec((1, 1, C), lambda b, nb: (b, 0, 0)),
        out_shape=jax.ShapeDtypeStruct((B, 1, C), jnp.float32),
    )(y, stats, g.reshape(1, C), bb.reshape(1, C))


def _head1_body(cnt, mid_ref, gd_ref, wmT_ref, wgT_ref, yo_ref, so_ref):
    b = pl.program_id(0)
    nb = pl.program_id(1)
    gvec = jnp.dot(gd_ref[0], wgT_ref[...],
                   preferred_element_type=jnp.float32, precision=jax.lax.Precision.HIGHEST)
    yo = jnp.dot(mid_ref[0, :, 0, :], wmT_ref[...],
                 preferred_element_type=jnp.float32, precision=jax.lax.Precision.HIGHEST) + gvec
    yo_ref[0, :, 0, :] = yo

    @pl.when((b == 0) & (nb == 0))
    def _():
        so_ref[...] = jnp.zeros_like(so_ref)

    so_ref[0:1, :] += jnp.sum(yo, axis=0, keepdims=True)
    so_ref[1:2, :] += jnp.sum(yo * yo, axis=0, keepdims=True)


def _head1(mid, gd, wmT, wgT, cnt):
    B, N, _, Cm = mid.shape
    Cg = wgT.shape[0]
    Cout = wgT.shape[1]
    grid = (B, N // _RB)
    return _pcall(
        functools.partial(_head1_body, cnt),
        grid,
        in_specs=[
            pl.BlockSpec((1, _RB, 1, Cm), lambda b, nb: (b, nb, 0, 0)),
            pl.BlockSpec((1, 1, Cg), lambda b, nb: (b, 0, 0)),
            pl.BlockSpec((Cm, Cout), lambda b, nb: (0, 0)),
            pl.BlockSpec((Cg, Cout), lambda b, nb: (0, 0)),
        ],
        out_specs=[
            pl.BlockSpec((1, _RB, 1, Cout), lambda b, nb: (b, nb, 0, 0)),
            pl.BlockSpec((8, Cout), lambda b, nb: (0, 0)),
        ],
        out_shape=[
            jax.ShapeDtypeStruct((B, N, 1, Cout), jnp.float32),
            jax.ShapeDtypeStruct((8, Cout), jnp.float32),
        ],
    )(mid, gd, wmT, wgT)


def kernel(x, W1, g1, b1, W2, g2, b2, W3, g3, b3, W4, g4, b4, W5, g5, b5,
           Wg, gg, bg, Wc1, gc1, bc1, Wc2, gc2, bc2, Wc3, bc3):
    B, C0, N = x.shape
    f0 = jnp.swapaxes(x, 1, 2)
    cnt_k = float(B * N * _K)
    cnt_n = float(B * N)

    def edge_block(f, Wa, Wb_, ga, ba, gb_, bb_):
        C = f.shape[2]
        wn = Wa[:, :C].T
        wcm = (Wa[:, C:] - Wa[:, :C]).T
        y1, s1 = _edge_pre(f, wn, wcm)
        y2, s2 = _conv_stage(y1, s1, ga, ba, Wb_.T, cnt_k)
        return _bn_max_k(y2, s2, gb_, bb_, cnt_k)

    x1 = edge_block(f0, W1, W2, g1, b1, g2, b2)
    x2 = edge_block(x1, W3, W4, g3, b3, g4, b4)

    C = x2.shape[2]
    y5, s5 = _edge_pre(x2, W5[:, :C].T, (W5[:, C:] - W5[:, :C]).T)
    x3 = _bn_max_k(y5, s5, g5, b5, cnt_k)

    mid = jnp.concatenate([x1, x2, x3], axis=2)
    mid4 = mid[:, :, None, :]

    yg, sg = _conv_stage(mid4, None, None, None, Wg.T, cnt_n)
    gd = _bn_max_n(yg, sg, gg, bg, cnt_n)

    Wc1g = Wc1[:, :1024]
    Wc1m = Wc1[:, 1024:1216] + Wc1[:, 1216:1408]
    yc1, sc1 = _head1(mid4, gd, Wc1m.T, Wc1g.T, cnt_n)
    yc2, sc2 = _conv_stage(yc1, sc1, gc1, bc1, Wc2.T, cnt_n)

    wc3p = jnp.zeros((256, 64), jnp.float32).at[:, :50].set(Wc3.T)
    bc3p = jnp.zeros((64,), jnp.float32).at[:50].set(bc3)
    yo, _ = _conv_stage(yc2, sc2, gc2, bc2, wc3p, cnt_n, bias=bc3p)
    out = yo[:, :, 0, :50]
    return jnp.swapaxes(out, 1, 2)


# speedup vs baseline: 2.7013x; 1.0009x over previous
import functools

import jax
import jax.numpy as jnp
from jax.experimental import pallas as pl

_K = 20
_EPS = 1e-5
_RB = 256
_NEG = -3.0e38

_INTERPRET = False
_HI = jax.lax.Precision.HIGHEST


def _pcall(body, grid, in_specs, out_specs, out_shape):
    return pl.pallas_call(body, grid=grid, in_specs=in_specs,
                          out_specs=out_specs, out_shape=out_shape,
                          interpret=_INTERPRET)


def _edge_pre_body(N, fb_ref, ff_ref, wnT_ref, wcmT_ref, y_ref, st_ref):
    b = pl.program_id(0)
    nb = pl.program_id(1)
    fb = fb_ref[0]
    ff = ff_ref[0]
    wnT = wnT_ref[...]
    wcmT = wcmT_ref[...]

    z = jnp.dot(ff, wnT, preferred_element_type=jnp.float32, precision=_HI)
    cterm = jnp.dot(fb, wcmT, preferred_element_type=jnp.float32, precision=_HI)

    sqf = jnp.sum(ff * ff, axis=1)[None, :]
    sqb = jnp.sum(fb * fb, axis=1, keepdims=True)
    inner = jax.lax.dot_general(fb, ff, (((1,), (1,)), ((), ())),
                                preferred_element_type=jnp.float32,
                                precision=_HI)
    pd = 2.0 * inner - sqb - sqf

    iota = jax.lax.broadcasted_iota(jnp.int32, (_RB, N), 1)
    rows = nb * _RB + jax.lax.broadcasted_iota(jnp.int32, (_RB, N), 0)
    pd = jnp.where(iota == rows, _NEG, pd)

    y0 = jnp.dot(fb, wnT, preferred_element_type=jnp.float32, precision=_HI) + cterm
    y_ref[0, :, 0, :] = y0
    s = jnp.sum(y0, axis=0, keepdims=True)
    s2 = jnp.sum(y0 * y0, axis=0, keepdims=True)

    for t in range(1, _K):
        m = jnp.max(pd, axis=1, keepdims=True)
        eq = pd >= m
        idxv = jnp.min(jnp.where(eq, iota, N), axis=1, keepdims=True)
        onehot = iota == idxv
        pd = jnp.where(onehot, _NEG, pd)
        oh = jnp.where(onehot, 1.0, 0.0).astype(jnp.float32)
        nbr = jnp.dot(oh, z, preferred_element_type=jnp.float32, precision=_HI)
        yt = nbr + cterm
        y_ref[0, :, t, :] = yt
        s = s + jnp.sum(yt, axis=0, keepdims=True)
        s2 = s2 + jnp.sum(yt * yt, axis=0, keepdims=True)

    @pl.when((b == 0) & (nb == 0))
    def _():
        st_ref[...] = jnp.zeros_like(st_ref)

    st_ref[0:1, :] += s
    st_ref[1:2, :] += s2


def _edge_pre(f, wnT, wcmT):
    B, N, C = f.shape
    O = wnT.shape[1]
    grid = (B, N // _RB)
    return _pcall(
        functools.partial(_edge_pre_body, N),
        grid,
        in_specs=[
            pl.BlockSpec((1, _RB, C), lambda b, nb: (b, nb, 0)),
            pl.BlockSpec((1, N, C), lambda b, nb: (b, 0, 0)),
            pl.BlockSpec((C, O), lambda b, nb: (0, 0)),
            pl.BlockSpec((C, O), lambda b, nb: (0, 0)),
        ],
        out_specs=[
            pl.BlockSpec((1, _RB, _K, O), lambda b, nb: (b, nb, 0, 0)),
            pl.BlockSpec((8, O), lambda b, nb: (0, 0)),
        ],
        out_shape=[
            jax.ShapeDtypeStruct((B, N, _K, O), jnp.float32),
            jax.ShapeDtypeStruct((8, O), jnp.float32),
        ],
    )(f, f, wnT, wcmT)


def _conv_stage_body(Kk, cnt, act, has_prev, has_bias, *refs):
    i = 0
    y_ref = refs[i]; i += 1
    if has_prev:
        st_in = refs[i]; i += 1
        g_ref = refs[i]; i += 1
        bb_ref = refs[i]; i += 1
    wT_ref = refs[i]; i += 1
    if has_bias:
        bias_ref = refs[i]; i += 1
    yo_ref = refs[i]; i += 1
    so_ref = refs[i]; i += 1

    b = pl.program_id(0)
    nb = pl.program_id(1)

    if has_prev:
        mean = st_in[0:1, :] / cnt
        var = st_in[1:2, :] / cnt - mean * mean
        scale = g_ref[...] * jax.lax.rsqrt(var + _EPS)
        shift = bb_ref[...] - mean * scale

    wT = wT_ref[...]
    s = None
    for k in range(Kk):
        h = y_ref[0, :, k, :]
        if has_prev:
            h = h * scale + shift
            if act:
                h = jnp.where(h >= 0, h, 0.2 * h)
        yo = jnp.dot(h, wT, preferred_element_type=jnp.float32, precision=_HI)
        if has_bias:
            yo = yo + bias_ref[...]
        yo_ref[0, :, k, :] = yo
        ds = jnp.sum(yo, axis=0, keepdims=True)
        ds2 = jnp.sum(yo * yo, axis=0, keepdims=True)
        s = ds if s is None else s + ds
        s2 = ds2 if k == 0 else s2 + ds2

    @pl.when((b == 0) & (nb == 0))
    def _():
        so_ref[...] = jnp.zeros_like(so_ref)

    so_ref[0:1, :] += s
    so_ref[1:2, :] += s2


def _conv_stage(y, stats_in, g, bb, wT, cnt, act=True, bias=None):
    B, N, Kk, Cin = y.shape
    Cout = wT.shape[1]
    grid = (B, N // _RB)
    has_prev = stats_in is not None
    has_bias = bias is not None
    in_specs = [pl.BlockSpec((1, _RB, Kk, Cin), lambda b, nb: (b, nb, 0, 0))]
    ops = [y]
    if has_prev:
        in_specs += [
            pl.BlockSpec((8, Cin), lambda b, nb: (0, 0)),
            pl.BlockSpec((1, Cin), lambda b, nb: (0, 0)),
            pl.BlockSpec((1, Cin), lambda b, nb: (0, 0)),
        ]
        ops += [stats_in, g.reshape(1, Cin), bb.reshape(1, Cin)]
    in_specs.append(pl.BlockSpec((Cin, Cout), lambda b, nb: (0, 0)))
    ops.append(wT)
    if has_bias:
        in_specs.append(pl.BlockSpec((1, Cout), lambda b, nb: (0, 0)))
        ops.append(bias.reshape(1, Cout))
    return _pcall(
        functools.partial(_conv_stage_body, Kk, cnt, act, has_prev, has_bias),
        grid,
        in_specs=in_specs,
        out_specs=[
            pl.BlockSpec((1, _RB, Kk, Cout), lambda b, nb: (b, nb, 0, 0)),
            pl.BlockSpec((8, Cout), lambda b, nb: (0, 0)),
        ],
        out_shape=[
            jax.ShapeDtypeStruct((B, N, Kk, Cout), jnp.float32),
            jax.ShapeDtypeStruct((8, Cout), jnp.float32),
        ],
    )(*ops)


def _bn_max_body(Kk, cnt, y_ref, st_ref, g_ref, b_ref, o_ref):
    mean = st_ref[0:1, :] / cnt
    var = st_ref[1:2, :] / cnt - mean * mean
    scale = g_ref[...] * jax.lax.rsqrt(var + _EPS)
    shift = b_ref[...] - mean * scale
    acc = None
    for k in range(Kk):
        h = y_ref[0, :, k, :] * scale + shift
        h = jnp.where(h >= 0, h, 0.2 * h)
        acc = h if acc is None else jnp.maximum(acc, h)
    o_ref[0] = acc


def _bn_max_k(y, stats, g, bb, cnt):
    B, N, Kk, C = y.shape
    grid = (B, N // _RB)
    return _pcall(
        functools.partial(_bn_max_body, Kk, cnt),
        grid,
        in_specs=[
            pl.BlockSpec((1, _RB, Kk, C), lambda b, nb: (b, nb, 0, 0)),
            pl.BlockSpec((8, C), lambda b, nb: (0, 0)),
            pl.BlockSpec((1, C), lambda b, nb: (0, 0)),
            pl.BlockSpec((1, C), lambda b, nb: (0, 0)),
        ],
        out_specs=pl.BlockSpec((1, _RB, C), lambda b, nb: (b, nb, 0)),
        out_shape=jax.ShapeDtypeStruct((B, N, C), jnp.float32),
    )(y, stats, g.reshape(1, C), bb.reshape(1, C))


def _bn_maxn_body(cnt, y_ref, st_ref, g_ref, b_ref, o_ref):
    nb = pl.program_id(1)
    mean = st_ref[0:1, :] / cnt
    var = st_ref[1:2, :] / cnt - mean * mean
    scale = g_ref[...] * jax.lax.rsqrt(var + _EPS)
    shift = b_ref[...] - mean * scale
    h = y_ref[0, :, 0, :] * scale + shift
    h = jnp.where(h >= 0, h, 0.2 * h)
    m = jnp.max(h, axis=0, keepdims=True)

    @pl.when(nb == 0)
    def _():
        o_ref[0] = m

    @pl.when(nb != 0)
    def _():
        o_ref[0] = jnp.maximum(o_ref[0], m)


def _bn_max_n(y, stats, g, bb, cnt):
    B, N, _, C = y.shape
    grid = (B, N // _RB)
    return _pcall(
        functools.partial(_bn_maxn_body, cnt),
        grid,
        in_specs=[
            pl.BlockSpec((1, _RB, 1, C), lambda b, nb: (b, nb, 0, 0)),
            pl.BlockSpec((8, C), lambda b, nb: (0, 0)),
            pl.BlockSpec((1, C), lambda b, nb: (0, 0)),
            pl.BlockSpec((1, C), lambda b, nb: (0, 0)),
        ],
        out_specs=pl.BlockSpec((1, 1, C), lambda b, nb: (b, 0, 0)),
        out_shape=jax.ShapeDtypeStruct((B, 1, C), jnp.float32),
    )(y, stats, g.reshape(1, C), bb.reshape(1, C))


def _head1_body(cnt, mid_ref, gd_ref, wmT_ref, wgT_ref, yo_ref, so_ref):
    b = pl.program_id(0)
    nb = pl.program_id(1)
    gvec = jnp.dot(gd_ref[0], wgT_ref[...],
                   preferred_element_type=jnp.float32, precision=_HI)
    yo = jnp.dot(mid_ref[0, :, 0, :], wmT_ref[...],
                 preferred_element_type=jnp.float32, precision=_HI) + gvec
    yo_ref[0, :, 0, :] = yo

    @pl.when((b == 0) & (nb == 0))
    def _():
        so_ref[...] = jnp.zeros_like(so_ref)

    so_ref[0:1, :] += jnp.sum(yo, axis=0, keepdims=True)
    so_ref[1:2, :] += jnp.sum(yo * yo, axis=0, keepdims=True)


def _head1(mid, gd, wmT, wgT, cnt):
    B, N, _, Cm = mid.shape
    Cg = wgT.shape[0]
    Cout = wgT.shape[1]
    grid = (B, N // _RB)
    return _pcall(
        functools.partial(_head1_body, cnt),
        grid,
        in_specs=[
            pl.BlockSpec((1, _RB, 1, Cm), lambda b, nb: (b, nb, 0, 0)),
            pl.BlockSpec((1, 1, Cg), lambda b, nb: (b, 0, 0)),
            pl.BlockSpec((Cm, Cout), lambda b, nb: (0, 0)),
            pl.BlockSpec((Cg, Cout), lambda b, nb: (0, 0)),
        ],
        out_specs=[
            pl.BlockSpec((1, _RB, 1, Cout), lambda b, nb: (b, nb, 0, 0)),
            pl.BlockSpec((8, Cout), lambda b, nb: (0, 0)),
        ],
        out_shape=[
            jax.ShapeDtypeStruct((B, N, 1, Cout), jnp.float32),
            jax.ShapeDtypeStruct((8, Cout), jnp.float32),
        ],
    )(mid, gd, wmT, wgT)


def kernel(x, W1, g1, b1, W2, g2, b2, W3, g3, b3, W4, g4, b4, W5, g5, b5,
           Wg, gg, bg, Wc1, gc1, bc1, Wc2, gc2, bc2, Wc3, bc3):
    B, C0, N = x.shape
    f0 = jnp.swapaxes(x, 1, 2)
    cnt_k = float(B * N * _K)
    cnt_n = float(B * N)

    def edge_block(f, Wa, Wb_, ga, ba, gb_, bb_):
        C = f.shape[2]
        wn = Wa[:, :C].T
        wcm = (Wa[:, C:] - Wa[:, :C]).T
        y1, s1 = _edge_pre(f, wn, wcm)
        y2, s2 = _conv_stage(y1, s1, ga, ba, Wb_.T, cnt_k)
        return _bn_max_k(y2, s2, gb_, bb_, cnt_k)

    x1 = edge_block(f0, W1, W2, g1, b1, g2, b2)
    x2 = edge_block(x1, W3, W4, g3, b3, g4, b4)

    C = x2.shape[2]
    y5, s5 = _edge_pre(x2, W5[:, :C].T, (W5[:, C:] - W5[:, :C]).T)
    x3 = _bn_max_k(y5, s5, g5, b5, cnt_k)

    mid = jnp.concatenate([x1, x2, x3], axis=2)
    mid4 = mid[:, :, None, :]

    yg, sg = _conv_stage(mid4, None, None, None, Wg.T, cnt_n)
    gd = _bn_max_n(yg, sg, gg, bg, cnt_n)

    Wc1g = Wc1[:, :1024]
    Wc1m = Wc1[:, 1024:1216] + Wc1[:, 1216:1408]
    yc1, sc1 = _head1(mid4, gd, Wc1m.T, Wc1g.T, cnt_n)
    yc2, sc2 = _conv_stage(yc1, sc1, gc1, bc1, Wc2.T, cnt_n)

    wc3p = jnp.zeros((256, 64), jnp.float32).at[:, :50].set(Wc3.T)
    bc3p = jnp.zeros((64,), jnp.float32).at[:50].set(bc3)
    yo, _ = _conv_stage(yc2, sc2, gc2, bc2, wc3p, cnt_n, bias=bc3p)
    out = yo[:, :, 0, :50]
    return jnp.swapaxes(out, 1, 2)
